# baseline (speedup 1.0000x reference)
import jax
import jax.numpy as jnp
from jax.experimental import pallas as pl
from jax.experimental.pallas import tpu as pltpu

N, NFEAT, NHID, DHID1 = 4096, 128, 64, 32
BM_ENC = 1024
BM_DEC = 512


def _enc_body(adj_ref, x_ref, wgc_ref, bgc_ref, w1_ref, b1_ref, w2_ref,
              b2_ref, w3_ref, b3_ref, wdec_ref, z_ref, zw_ref):
    xw = jnp.dot(x_ref[...], wgc_ref[...], preferred_element_type=jnp.float32)
    h = jnp.dot(adj_ref[...], xw, preferred_element_type=jnp.float32)
    h = jnp.maximum(h + bgc_ref[...], 0.0)
    h = jnp.maximum(jnp.dot(h, w1_ref[...],
                            preferred_element_type=jnp.float32) + b1_ref[...], 0.0)
    h = jnp.maximum(jnp.dot(h, w2_ref[...],
                            preferred_element_type=jnp.float32) + b2_ref[...], 0.0)
    z = jnp.dot(h, w3_ref[...], preferred_element_type=jnp.float32) + b3_ref[...]
    z_ref[...] = z
    zw_ref[...] = jnp.dot(z, wdec_ref[...], preferred_element_type=jnp.float32)


def _dec_body(zw_ref, z_ref, out_ref):
    out_ref[...] = jax.lax.dot_general(
        zw_ref[...], z_ref[...], (((1,), (1,)), ((), ())),
        preferred_element_type=jnp.float32)


@jax.jit
def kernel(x, adj_norm_pos, W_gc, b_gc, W1, b1, W2, b2, W3, b3, W_dec):
    full = lambda shape: pl.BlockSpec(shape, lambda i: (0,) * len(shape))

    z, zw = pl.pallas_call(
        _enc_body,
        grid=(N // BM_ENC,),
        in_specs=[
            pl.BlockSpec((BM_ENC, N), lambda i: (i, 0)),
            full((N, NFEAT)),
            full((NFEAT, NHID)),
            full((1, NHID)),
            full((NHID, DHID1)),
            full((1, DHID1)),
            full((DHID1, 2 * DHID1)),
            full((1, 2 * DHID1)),
            full((2 * DHID1, DHID1)),
            full((1, DHID1)),
            full((DHID1, DHID1)),
        ],
        out_specs=[
            pl.BlockSpec((BM_ENC, DHID1), lambda i: (i, 0)),
            pl.BlockSpec((BM_ENC, DHID1), lambda i: (i, 0)),
        ],
        out_shape=[
            jax.ShapeDtypeStruct((N, DHID1), jnp.float32),
            jax.ShapeDtypeStruct((N, DHID1), jnp.float32),
        ],
        compiler_params=pltpu.CompilerParams(
            dimension_semantics=("parallel",)),
    )(adj_norm_pos, x, W_gc, b_gc.reshape(1, -1), W1, b1.reshape(1, -1),
      W2, b2.reshape(1, -1), W3, b3.reshape(1, -1), W_dec)

    logits = pl.pallas_call(
        _dec_body,
        grid=(N // BM_DEC,),
        in_specs=[
            pl.BlockSpec((BM_DEC, DHID1), lambda i: (i, 0)),
            full((N, DHID1)),
        ],
        out_specs=pl.BlockSpec((BM_DEC, N), lambda i: (i, 0)),
        out_shape=jax.ShapeDtypeStruct((N, N), jnp.float32),
        compiler_params=pltpu.CompilerParams(
            dimension_semantics=("parallel",)),
    )(zw, z)
    return logits


# speedup vs baseline: 1.0721x; 1.0721x over previous
import jax
import jax.numpy as jnp
from jax.experimental import pallas as pl
from jax.experimental.pallas import tpu as pltpu

N, NFEAT, NHID, DHID1 = 4096, 128, 64, 32
BM_ENC = 1024
KS = 2
BK = N // KS
BM_DEC = 512
NR = N // BM_ENC
NE = NR * KS
ND = N // BM_DEC


def _body(adj_ref, x_ref, wgc_ref, bgc_ref, w1_ref, b1_ref, w2_ref,
          b2_ref, w3_ref, b3_ref, wdec_ref, out_ref, xw_scr, z_scr, zw_scr,
          hacc_scr):
    i = pl.program_id(0)

    @pl.when(i == 0)
    def _():
        xw_scr[...] = jnp.dot(x_ref[...], wgc_ref[...],
                              preferred_element_type=jnp.float32)

    @pl.when(i < NE)
    def _():
        r = i // KS
        k = i % KS
        part = jnp.dot(adj_ref[...], xw_scr[pl.ds(k * BK, BK), :],
                       preferred_element_type=jnp.float32)

        @pl.when(k == 0)
        def _():
            hacc_scr[...] = part

        @pl.when(k > 0)
        def _():
            hacc_scr[...] = hacc_scr[...] + part

        @pl.when(k == KS - 1)
        def _():
            h = jnp.maximum(hacc_scr[...] + bgc_ref[...], 0.0)
            h = jnp.maximum(jnp.dot(h, w1_ref[...],
                                    preferred_element_type=jnp.float32)
                            + b1_ref[...], 0.0)
            h = jnp.maximum(jnp.dot(h, w2_ref[...],
                                    preferred_element_type=jnp.float32)
                            + b2_ref[...], 0.0)
            z = (jnp.dot(h, w3_ref[...], preferred_element_type=jnp.float32)
                 + b3_ref[...])
            z_scr[pl.ds(r * BM_ENC, BM_ENC), :] = z
            zw_scr[pl.ds(r * BM_ENC, BM_ENC), :] = jnp.dot(
                z, wdec_ref[...], preferred_element_type=jnp.float32)

    @pl.when(i >= NE)
    def _():
        j = i - NE
        out_ref[...] = jax.lax.dot_general(
            zw_scr[pl.ds(j * BM_DEC, BM_DEC), :], z_scr[...],
            (((1,), (1,)), ((), ())), preferred_element_type=jnp.float32)


def _adj_index(i):
    r = jax.lax.min(i // KS, NR - 1)
    k = jax.lax.select(i < NE, i % KS, KS - 1)
    return (r, k)


@jax.jit
def kernel(x, adj_norm_pos, W_gc, b_gc, W1, b1, W2, b2, W3, b3, W_dec):
    full = lambda shape: pl.BlockSpec(shape, lambda i: (0,) * len(shape))

    logits = pl.pallas_call(
        _body,
        grid=(NE + ND,),
        in_specs=[
            pl.BlockSpec((BM_ENC, BK), _adj_index),
            full((N, NFEAT)),
            full((NFEAT, NHID)),
            full((1, NHID)),
            full((NHID, DHID1)),
            full((1, DHID1)),
            full((DHID1, 2 * DHID1)),
            full((1, 2 * DHID1)),
            full((2 * DHID1, DHID1)),
            full((1, DHID1)),
            full((DHID1, DHID1)),
        ],
        out_specs=pl.BlockSpec((BM_DEC, N), lambda i: (jax.lax.max(i - NE, 0), 0)),
        out_shape=jax.ShapeDtypeStruct((N, N), jnp.float32),
        scratch_shapes=[
            pltpu.VMEM((N, NHID), jnp.float32),
            pltpu.VMEM((N, DHID1), jnp.float32),
            pltpu.VMEM((N, DHID1), jnp.float32),
            pltpu.VMEM((BM_ENC, NHID), jnp.float32),
        ],
        compiler_params=pltpu.CompilerParams(
            dimension_semantics=("arbitrary",)),
    )(adj_norm_pos, x, W_gc, b_gc.reshape(1, -1), W1, b1.reshape(1, -1),
      W2, b2.reshape(1, -1), W3, b3.reshape(1, -1), W_dec)
    return logits
